# baseline (speedup 1.0000x reference)
import functools

import jax
import jax.numpy as jnp
from jax import lax
from jax.experimental import pallas as pl
from jax.experimental.pallas import tpu as pltpu
from jax.experimental.pallas import tpu_sc as plsc
from jax.experimental import layout as jax_layout

_B, _L, _D = 4096, 200, 64
_V = 1000000
_NC, _NS = 2, 16
_NW = _NC * _NS
_TPW = _B * _L // _NW
_POS = (10, 12, 13, 14, 43, 44)
_NEG = (11, 15, 45, 46)
_PBLK = 8000


_CH = 27776
_NFULL = 36
_VPAD = _V + 64
_NCH = _NFULL + 1


def _tc_project(table, fcw):
  def body(w_ref, tail_ref, t_hbm, o_hbm, b0, b1, o0, o1,
           is0, is1, os0, os1):
    i = pl.program_id(0)
    bufs, obufs = (b0, b1), (o0, o1)
    isems, osems = (is0, is1), (os0, os1)
    w = w_ref[...]

    def start_in(j, buf, sem):
      @pl.when(j < _NFULL)
      def _():
        off = pl.multiple_of(j * _CH, 128)
        pltpu.async_copy(t_hbm.at[:, pl.ds(off, _CH)], buf, sem)

    def wait_in(j, buf, sem):
      @pl.when(j < _NFULL)
      def _():
        pltpu.make_async_copy(t_hbm.at[:, pl.ds(0, _CH)], buf, sem).wait()

    def start_out(j, obuf, sem):
      @pl.when(j < _NFULL)
      def _():
        off = pl.multiple_of(j * _CH, 128)
        pltpu.async_copy(obuf, o_hbm.at[pl.ds(off, _CH)], sem)

      @pl.when(j == _NFULL)
      def _():
        pltpu.async_copy(obuf.at[pl.ds(0, 128)],
                         o_hbm.at[pl.ds(_NFULL * _CH, 128)], sem)

    def wait_out(j, obuf, sem):
      @pl.when(j < _NFULL)
      def _():
        pltpu.make_async_copy(obuf, o_hbm.at[pl.ds(0, _CH)], sem).wait()

      @pl.when(j == _NFULL)
      def _():
        pltpu.make_async_copy(obuf.at[pl.ds(0, 128)],
                              o_hbm.at[pl.ds(0, 128)], sem).wait()

    def step(par):
      j = i
      buf, isem = bufs[par], isems[par]
      obuf, osem = obufs[par], osems[par]

      @pl.when(j == 0)
      def _():
        start_in(0, buf, isem)

      @pl.when(j + 1 < _NCH)
      def _():
        start_in(j + 1, bufs[par ^ 1], isems[par ^ 1])

      wait_in(j, buf, isem)

      @pl.when(j >= 2)
      def _():
        wait_out(j - 2, obuf, osem)

      @pl.when(j < _NFULL)
      def _():
        m = buf[...] * w
        m = m[0:32] + m[32:64]
        m = m[0:16] + m[16:32]
        m = m[0:8] + m[8:16]
        obuf[...] = jnp.sum(m, axis=0)

      @pl.when(j == _NFULL)
      def _():
        obuf[pl.ds(0, _D)] = jnp.sum(tail_ref[...] * w, axis=0)

      start_out(j, obuf, osem)

      @pl.when(j == _NCH - 1)
      def _():
        wait_out(j - 1, obufs[par ^ 1], osems[par ^ 1])
        wait_out(j, obuf, osem)

    @pl.when(i % 2 == 0)
    def _():
      step(0)

    @pl.when(i % 2 == 1)
    def _():
      step(1)

  tT = table.T
  tail = jax.lax.slice(tT, (0, _NFULL * _CH), (_D, _V))
  return pl.pallas_call(
      body,
      grid=(_NCH,),
      in_specs=[
          pl.BlockSpec((_D, 1), lambda i: (0, 0)),
          pl.BlockSpec((_D, _D), lambda i: (0, 0)),
          pl.BlockSpec(memory_space=pltpu.HBM),
      ],
      out_specs=pl.BlockSpec(memory_space=pltpu.HBM),
      out_shape=jax.ShapeDtypeStruct((_VPAD,), jnp.float32),
      scratch_shapes=[
          pltpu.VMEM((_D, _CH), jnp.float32),
          pltpu.VMEM((_D, _CH), jnp.float32),
          pltpu.VMEM((_CH,), jnp.float32),
          pltpu.VMEM((_CH,), jnp.float32),
          pltpu.SemaphoreType.DMA,
          pltpu.SemaphoreType.DMA,
          pltpu.SemaphoreType.DMA,
          pltpu.SemaphoreType.DMA,
      ],
  )(fcw.reshape(_D, 1), tail, tT)


def _sc_gather(ids_flat, p):
  mesh = plsc.VectorSubcoreMesh(core_axis_name="c", subcore_axis_name="s")

  @functools.partial(
      pl.kernel,
      out_type=jax.ShapeDtypeStruct((_B * _L,), jnp.float32),
      mesh=mesh,
      compiler_params=pltpu.CompilerParams(use_tc_tiling_on_sc=False),
      scratch_types=[
          pltpu.VMEM_SHARED((_VPAD,), jnp.float32),
          pltpu.VMEM((_TPW,), jnp.int32),
          pltpu.VMEM((_TPW,), jnp.float32),
          pltpu.SemaphoreType.DMA,
      ],
  )
  def g(ids_hbm, p_hbm, out_hbm, p_s, ids_v, vals_v, sem):
    cid = lax.axis_index("c")
    sid = lax.axis_index("s")
    wid = sid * _NC + cid
    tokbase = pl.multiple_of(wid * _TPW, 8)

    pltpu.sync_copy(ids_hbm.at[pl.ds(tokbase, _TPW)], ids_v)

    @pl.when(sid == 0)
    def _():
      pltpu.sync_copy(p_hbm, p_s)

    plsc.subcore_barrier()

    def g_body(j, carry):
      o = pl.multiple_of(j * 128, 8)
      pltpu.async_copy(p_s.at[ids_v.at[pl.ds(o, 128)]],
                       vals_v.at[pl.ds(o, 128)], sem)
      return carry
    lax.fori_loop(0, _TPW // 128, g_body, 0)

    pltpu.make_async_copy(p_hbm.at[pl.ds(0, _TPW)], vals_v, sem).wait()

    pltpu.sync_copy(vals_v, out_hbm.at[pl.ds(tokbase, _TPW)])

  return g(ids_flat, p)


_CBLK = 512
_CGRID = _B // _CBLK


def _tc_bonus(input_ids):
  def body(ids_ref, o_ref):
    i = pl.program_id(0)
    ids = ids_ref[...]
    bonus = jnp.zeros((_CBLK,), jnp.float32)
    for c in _POS:
      bonus = bonus + 0.5 * jnp.any(ids == c, axis=1).astype(jnp.float32)
    for c in _NEG:
      bonus = bonus - 0.5 * jnp.any(ids == c, axis=1).astype(jnp.float32)
    o_ref[pl.ds(i * _CBLK, _CBLK)] = bonus

  return pl.pallas_call(
      body,
      grid=(_CGRID,),
      in_specs=[pl.BlockSpec((_CBLK, _L), lambda i: (i, 0))],
      out_specs=pl.BlockSpec(memory_space=pltpu.VMEM),
      out_shape=jax.ShapeDtypeStruct((_B,), jnp.float32),
  )(input_ids)


def _tc_finish(attention_mask, vals, bonus, fc_b):
  def body(m_ref, v_ref, bon_ref, b_ref, o_ref):
    i = pl.program_id(0)
    m = m_ref[...].astype(jnp.float32)
    cnt = jnp.maximum(jnp.sum(m, axis=1), 1e-8)
    num = jnp.sum(v_ref[...] * m, axis=1)
    bon = bon_ref[pl.ds(i * _CBLK, _CBLK)]
    o_ref[pl.ds(i * _CBLK, _CBLK)] = num / cnt + b_ref[0] + bon

  return pl.pallas_call(
      body,
      grid=(_CGRID,),
      in_specs=[
          pl.BlockSpec((_CBLK, _L), lambda i: (i, 0)),
          pl.BlockSpec((_CBLK, _L), lambda i: (i, 0)),
          pl.BlockSpec(memory_space=pltpu.VMEM),
          pl.BlockSpec(memory_space=pltpu.SMEM),
      ],
      out_specs=pl.BlockSpec(memory_space=pltpu.VMEM),
      out_shape=jax.ShapeDtypeStruct((_B,), jnp.float32),
  )(attention_mask, vals, bonus, fc_b)


@jax.jit
def kernel(input_ids, attention_mask, table, fc_w, fc_b):
  fcw = fc_w.reshape(-1).astype(jnp.float32)
  p = _tc_project(table, fcw)
  vals = _sc_gather(input_ids.reshape(-1), p)
  bonus = _tc_bonus(input_ids)
  return _tc_finish(attention_mask, vals.reshape(_B, _L), bonus, fc_b)


# speedup vs baseline: 1.0453x; 1.0453x over previous
import functools

import jax
import jax.numpy as jnp
from jax import lax
from jax.experimental import pallas as pl
from jax.experimental.pallas import tpu as pltpu
from jax.experimental.pallas import tpu_sc as plsc
from jax.experimental import layout as jax_layout

_B, _L, _D = 4096, 200, 64
_V = 1000000
_NC, _NS = 2, 16
_NW = _NC * _NS
_TPW = _B * _L // _NW
_POS = (10, 12, 13, 14, 43, 44)
_NEG = (11, 15, 45, 46)
_PBLK = 8000


_CH = 55552
_NFULL = 18
_VPAD = _V + 64
_NCH = _NFULL + 1


def _tc_project(table, fcw):
  def body(w_ref, tail_ref, t_hbm, o_hbm, b0, b1, o0, o1,
           is0, is1, os0, os1):
    i = pl.program_id(0)
    bufs, obufs = (b0, b1), (o0, o1)
    isems, osems = (is0, is1), (os0, os1)
    w = w_ref[...]

    def start_in(j, buf, sem):
      @pl.when(j < _NFULL)
      def _():
        off = pl.multiple_of(j * _CH, 128)
        pltpu.async_copy(t_hbm.at[:, pl.ds(off, _CH)], buf, sem)

    def wait_in(j, buf, sem):
      @pl.when(j < _NFULL)
      def _():
        pltpu.make_async_copy(t_hbm.at[:, pl.ds(0, _CH)], buf, sem).wait()

    def start_out(j, obuf, sem):
      @pl.when(j < _NFULL)
      def _():
        off = pl.multiple_of(j * _CH, 128)
        pltpu.async_copy(obuf, o_hbm.at[pl.ds(off, _CH)], sem)

      @pl.when(j == _NFULL)
      def _():
        pltpu.async_copy(obuf.at[pl.ds(0, 128)],
                         o_hbm.at[pl.ds(_NFULL * _CH, 128)], sem)

    def wait_out(j, obuf, sem):
      @pl.when(j < _NFULL)
      def _():
        pltpu.make_async_copy(obuf, o_hbm.at[pl.ds(0, _CH)], sem).wait()

      @pl.when(j == _NFULL)
      def _():
        pltpu.make_async_copy(obuf.at[pl.ds(0, 128)],
                              o_hbm.at[pl.ds(0, 128)], sem).wait()

    def step(par):
      j = i
      buf, isem = bufs[par], isems[par]
      obuf, osem = obufs[par], osems[par]

      @pl.when(j == 0)
      def _():
        start_in(0, buf, isem)

      @pl.when(j + 1 < _NCH)
      def _():
        start_in(j + 1, bufs[par ^ 1], isems[par ^ 1])

      wait_in(j, buf, isem)

      @pl.when(j >= 2)
      def _():
        wait_out(j - 2, obuf, osem)

      @pl.when(j < _NFULL)
      def _():
        obuf[...] = jnp.sum(buf[...] * w, axis=0)

      @pl.when(j == _NFULL)
      def _():
        obuf[pl.ds(0, _D)] = jnp.sum(tail_ref[...] * w, axis=0)

      start_out(j, obuf, osem)

      @pl.when(j == _NCH - 1)
      def _():
        wait_out(j - 1, obufs[par ^ 1], osems[par ^ 1])
        wait_out(j, obuf, osem)

    @pl.when(i % 2 == 0)
    def _():
      step(0)

    @pl.when(i % 2 == 1)
    def _():
      step(1)

  tT = table.T
  tail = jax.lax.slice(tT, (0, _NFULL * _CH), (_D, _V))
  return pl.pallas_call(
      body,
      grid=(_NCH,),
      in_specs=[
          pl.BlockSpec((_D, 1), lambda i: (0, 0)),
          pl.BlockSpec((_D, _D), lambda i: (0, 0)),
          pl.BlockSpec(memory_space=pltpu.HBM),
      ],
      out_specs=pl.BlockSpec(memory_space=pltpu.HBM),
      out_shape=jax.ShapeDtypeStruct((_VPAD,), jnp.float32),
      scratch_shapes=[
          pltpu.VMEM((_D, _CH), jnp.float32),
          pltpu.VMEM((_D, _CH), jnp.float32),
          pltpu.VMEM((_CH,), jnp.float32),
          pltpu.VMEM((_CH,), jnp.float32),
          pltpu.SemaphoreType.DMA,
          pltpu.SemaphoreType.DMA,
          pltpu.SemaphoreType.DMA,
          pltpu.SemaphoreType.DMA,
      ],
  )(fcw.reshape(_D, 1), tail, tT)


def _sc_gather(ids_flat, p):
  mesh = plsc.VectorSubcoreMesh(core_axis_name="c", subcore_axis_name="s")

  @functools.partial(
      pl.kernel,
      out_type=jax.ShapeDtypeStruct((_B * _L,), jnp.float32),
      mesh=mesh,
      compiler_params=pltpu.CompilerParams(use_tc_tiling_on_sc=False),
      scratch_types=[
          pltpu.VMEM_SHARED((_VPAD,), jnp.float32),
          pltpu.VMEM((_TPW,), jnp.int32),
          pltpu.VMEM((_TPW,), jnp.float32),
          pltpu.SemaphoreType.DMA,
      ],
  )
  def g(ids_hbm, p_hbm, out_hbm, p_s, ids_v, vals_v, sem):
    cid = lax.axis_index("c")
    sid = lax.axis_index("s")
    wid = sid * _NC + cid
    tokbase = pl.multiple_of(wid * _TPW, 8)

    pltpu.sync_copy(ids_hbm.at[pl.ds(tokbase, _TPW)], ids_v)

    @pl.when(sid == 0)
    def _():
      pltpu.sync_copy(p_hbm, p_s)

    plsc.subcore_barrier()

    def g_body(j, carry):
      o = pl.multiple_of(j * 128, 8)
      pltpu.async_copy(p_s.at[ids_v.at[pl.ds(o, 128)]],
                       vals_v.at[pl.ds(o, 128)], sem)
      return carry
    lax.fori_loop(0, _TPW // 128, g_body, 0)

    pltpu.make_async_copy(p_hbm.at[pl.ds(0, _TPW)], vals_v, sem).wait()

    pltpu.sync_copy(vals_v, out_hbm.at[pl.ds(tokbase, _TPW)])

  return g(ids_flat, p)


_CBLK = 512
_CGRID = _B // _CBLK


def _tc_bonus(input_ids):
  def body(ids_ref, o_ref):
    i = pl.program_id(0)
    ids = ids_ref[...]
    bonus = jnp.zeros((_CBLK,), jnp.float32)
    for c in _POS:
      bonus = bonus + 0.5 * jnp.any(ids == c, axis=1).astype(jnp.float32)
    for c in _NEG:
      bonus = bonus - 0.5 * jnp.any(ids == c, axis=1).astype(jnp.float32)
    o_ref[pl.ds(i * _CBLK, _CBLK)] = bonus

  return pl.pallas_call(
      body,
      grid=(_CGRID,),
      in_specs=[pl.BlockSpec((_CBLK, _L), lambda i: (i, 0))],
      out_specs=pl.BlockSpec(memory_space=pltpu.VMEM),
      out_shape=jax.ShapeDtypeStruct((_B,), jnp.float32),
  )(input_ids)


def _tc_finish(attention_mask, vals, bonus, fc_b):
  def body(m_ref, v_ref, bon_ref, b_ref, o_ref):
    i = pl.program_id(0)
    m = m_ref[...].astype(jnp.float32)
    cnt = jnp.maximum(jnp.sum(m, axis=1), 1e-8)
    num = jnp.sum(v_ref[...] * m, axis=1)
    bon = bon_ref[pl.ds(i * _CBLK, _CBLK)]
    o_ref[pl.ds(i * _CBLK, _CBLK)] = num / cnt + b_ref[0] + bon

  return pl.pallas_call(
      body,
      grid=(_CGRID,),
      in_specs=[
          pl.BlockSpec((_CBLK, _L), lambda i: (i, 0)),
          pl.BlockSpec((_CBLK, _L), lambda i: (i, 0)),
          pl.BlockSpec(memory_space=pltpu.VMEM),
          pl.BlockSpec(memory_space=pltpu.SMEM),
      ],
      out_specs=pl.BlockSpec(memory_space=pltpu.VMEM),
      out_shape=jax.ShapeDtypeStruct((_B,), jnp.float32),
  )(attention_mask, vals, bonus, fc_b)


@jax.jit
def kernel(input_ids, attention_mask, table, fc_w, fc_b):
  fcw = fc_w.reshape(-1).astype(jnp.float32)
  p = _tc_project(table, fcw)
  vals = _sc_gather(input_ids.reshape(-1), p)
  bonus = _tc_bonus(input_ids)
  return _tc_finish(attention_mask, vals.reshape(_B, _L), bonus, fc_b)
